# baseline (speedup 1.0000x reference)
import functools

import jax
import jax.numpy as jnp
from jax import lax
from jax.experimental import pallas as pl
from jax.experimental.pallas import tpu as pltpu
from jax.experimental.pallas import tpu_sc as plsc

_NUM_ENTITY = 1000000
_DIM = 32
_BATCH = 16384

_info = plsc.get_sparse_core_info()
_NC = _info.num_cores
_NS = _info.num_subcores
_L = _info.num_lanes
_NW = _NC * _NS
_BPW = _BATCH // _NW
_CHUNK = 128
_NCHUNK = _BPW // _CHUNK
_GROUPS = _BPW // _L

_mesh = plsc.VectorSubcoreMesh(core_axis_name="c", subcore_axis_name="s")


@functools.partial(
    pl.kernel,
    mesh=_mesh,
    compiler_params=pltpu.CompilerParams(
        needs_layout_passes=False, use_tc_tiling_on_sc=False),
    out_type=jax.ShapeDtypeStruct((_BATCH,), jnp.float32),
    scratch_types=[
        pltpu.VMEM((_NCHUNK, _CHUNK), jnp.int32),
        pltpu.VMEM((_NCHUNK, _CHUNK), jnp.int32),
        pltpu.VMEM((_BPW, _DIM), jnp.float32),
        pltpu.VMEM((_BPW, _DIM), jnp.float32),
        pltpu.VMEM((_BPW,), jnp.float32),
        pltpu.SemaphoreType.DMA,
        pltpu.SemaphoreType.DMA,
    ],
)
def _sc_pair_dot(idx0_hbm, idx1_hbm, win_hbm, wout_hbm, out_hbm,
                 idx0_v, idx1_v, in_v, outr_v, res_v, sem_a, sem_b):
    wid = lax.axis_index("s") * _NC + lax.axis_index("c")
    base_chunk = wid * _NCHUNK

    pltpu.sync_copy(idx0_hbm.at[pl.ds(base_chunk, _NCHUNK)], idx0_v)
    pltpu.sync_copy(idx1_hbm.at[pl.ds(base_chunk, _NCHUNK)], idx1_v)

    copies = []
    for k in range(_NCHUNK):
        sl = pl.ds(k * _CHUNK, _CHUNK)
        copies.append(pltpu.async_copy(
            win_hbm.at[idx0_v.at[k]], in_v.at[sl], sem_a))
        copies.append(pltpu.async_copy(
            wout_hbm.at[idx1_v.at[k]], outr_v.at[sl], sem_b))
    for cp in copies:
        cp.wait()

    lanes = lax.iota(jnp.int32, _L)

    def group_body(g, carry):
        rows = lanes + g * _L
        acc = jnp.zeros((_L,), jnp.float32)
        for d in range(_DIM):
            col = jnp.full((_L,), d, jnp.int32)
            a = plsc.load_gather(in_v, [rows, col])
            b = plsc.load_gather(outr_v, [rows, col])
            acc = acc + a * b
        plsc.store_scatter(res_v, [rows], acc)
        return carry

    lax.fori_loop(0, _GROUPS, group_body, 0)

    pltpu.sync_copy(res_v, out_hbm.at[pl.ds(wid * _BPW, _BPW)])


def kernel(idxs, W_in, W_out):
    idx0 = idxs[:, 0].reshape(_BATCH // _CHUNK, _CHUNK).astype(jnp.int32)
    idx1 = idxs[:, 1].reshape(_BATCH // _CHUNK, _CHUNK).astype(jnp.int32)
    return _sc_pair_dot(idx0, idx1, W_in, W_out)
